# baseline (speedup 1.0000x reference)
import functools

import jax
import jax.numpy as jnp
from jax import lax
from jax.experimental import pallas as pl
from jax.experimental.pallas import tpu as pltpu
from jax.experimental.pallas import tpu_sc as plsc

TASK_NUMS = 1000
BLOCK_ROWS = 100000
D = 64
BATCH = 16384
NC = 2
NS = 16
NW = NC * NS
TASK_PER_W = BATCH // NW
HALF = TASK_PER_W // 2
GATHER_CHUNK = 128

BLOCK_CHUNK = 168
BLOCK_CHUNKS_PER_W = 19
BLOCK_LAST_START = BLOCK_ROWS - BLOCK_CHUNK


def _make_kernel():
    mesh = plsc.VectorSubcoreMesh(core_axis_name="c", subcore_axis_name="s")

    @functools.partial(
        pl.kernel,
        mesh=mesh,
        out_type=jax.ShapeDtypeStruct((BATCH + BLOCK_ROWS, D), jnp.float32),
        scratch_types=[
            pltpu.VMEM((TASK_PER_W,), jnp.int32),
            pltpu.VMEM((4, GATHER_CHUNK), jnp.int32),
            pltpu.VMEM((HALF, 2 * D), jnp.float32),
            pltpu.VMEM((HALF, D), jnp.float32),
            pltpu.VMEM((BLOCK_CHUNK, D), jnp.float32),
            pltpu.VMEM((BLOCK_CHUNK, D), jnp.float32),
            pltpu.SemaphoreType.DMA,
            pltpu.SemaphoreType.DMA,
            pltpu.SemaphoreType.DMA,
        ],
        compiler_params=pltpu.CompilerParams(needs_layout_passes=False),
    )
    def k(idx_hbm, task_pairs_hbm, block_w_hbm, out_hbm,
          idx_v, pidx_v, prow_v, comp_v, blk_a, blk_b,
          gsem, rsem, wsem):
        wid = lax.axis_index("s") * NC + lax.axis_index("c")
        tbase = wid * TASK_PER_W

        def chunk_start(j):
            return pl.multiple_of(
                jnp.minimum((wid * BLOCK_CHUNKS_PER_W + j) * BLOCK_CHUNK,
                            BLOCK_LAST_START), 8)

        bufs = (blk_a, blk_b)

        reads = [pltpu.async_copy(
            block_w_hbm.at[pl.ds(chunk_start(0), BLOCK_CHUNK)], blk_a, rsem)]

        pltpu.sync_copy(idx_hbm.at[pl.ds(tbase, TASK_PER_W)], idx_v)
        for g in range(TASK_PER_W // 16):
            pidx_v[g // 8, pl.ds((g % 8) * 16, 16)] = \
                idx_v[pl.ds(g * 16, 16)] >> 1

        def do_half(h):
            gathers = [
                pltpu.async_copy(
                    task_pairs_hbm.at[pidx_v.at[2 * h + j]],
                    prow_v.at[pl.ds(j * GATHER_CHUNK, GATHER_CHUNK)],
                    gsem)
                for j in range(2)
            ]
            for g in gathers:
                g.wait()

            def compact(g, _):
                rows = lax.broadcasted_iota(jnp.int32, (16,), 0) + g * 16
                hoff = (idx_v[pl.ds(h * HALF + g * 16, 16)] & 1) * D
                for c in range(D):
                    v = plsc.load_gather(prow_v, [rows, hoff + c])
                    plsc.store_scatter(
                        comp_v, [rows, jnp.full((16,), c, jnp.int32)], v)
                return 0

            lax.fori_loop(0, HALF // 16, compact, 0)
            pltpu.sync_copy(comp_v,
                            out_hbm.at[pl.ds(tbase + h * HALF, HALF)])

        do_half(0)
        do_half(1)

        writes = [None] * BLOCK_CHUNKS_PER_W
        for j in range(BLOCK_CHUNKS_PER_W):
            if j + 1 < BLOCK_CHUNKS_PER_W:
                if j - 1 >= 0:
                    writes[j - 1].wait()
                reads.append(pltpu.async_copy(
                    block_w_hbm.at[pl.ds(chunk_start(j + 1), BLOCK_CHUNK)],
                    bufs[(j + 1) % 2], rsem))
            reads[j].wait()
            writes[j] = pltpu.async_copy(
                bufs[j % 2],
                out_hbm.at[pl.ds(BATCH + chunk_start(j), BLOCK_CHUNK)],
                wsem)

        writes[-2].wait()
        writes[-1].wait()

    return k


_sc_kernel = _make_kernel()


def kernel(task_ids, task_embs_weight, block_emb_weight, block_emb_input):
    del block_emb_input
    task_pairs = task_embs_weight.reshape(TASK_NUMS // 2, 2 * D)
    return _sc_kernel(task_ids, task_pairs, block_emb_weight)
